# baseline (speedup 1.0000x reference)
import functools

import jax
import jax.numpy as jnp
from jax import lax
from jax.experimental import pallas as pl
from jax.experimental.pallas import tpu as pltpu
from jax.experimental.pallas import tpu_sc as plsc

NC = 2
NS = 16
NW = NC * NS
L = 16

CHUNK = 80
GROUPS = CHUNK // L
IBLK = 25
FLUSH = 10
STRIPE = 1000


def _tec_body(D, per_w, n_nodes, table_hbm, cidx_hbm, out_hbm,
              table_sh, cblk0, cblk1, rows0, rows1,
              acc_v, out_v, sem0, sem1, bsem0, bsem1):
  wid = lax.axis_index("s") * NC + lax.axis_index("c")
  sid = lax.axis_index("s")
  n_chunks = per_w // CHUNK
  n_blocks = n_chunks // IBLK
  base = wid * per_w
  cbase = wid * (per_w * 2)
  blk_words = IBLK * CHUNK * 2
  nslice = D // L
  gbufs = ((rows0, sem0), (rows1, sem1))
  iblks = ((cblk0, bsem0), (cblk1, bsem1))

  @pl.when(sid < n_nodes // STRIPE)
  def _():
    off = pl.multiple_of(sid * STRIPE, 8)
    pltpu.sync_copy(table_hbm.at[pl.ds(off, STRIPE)], table_sh.at[pl.ds(off, STRIPE)])

  plsc.subcore_barrier()

  def blk_ref(bj):
    off = pl.multiple_of(cbase + bj * blk_words, 8)
    return cidx_hbm.at[pl.ds(off, blk_words)]

  def _for_parity(ci, fn):
    p = lax.rem(ci // IBLK, 2)

    @pl.when(p == 0)
    def _():
      fn(0)

    @pl.when(p == 1)
    def _():
      fn(1)

  def fire_blk(bj):
    def go(s):
      cblk, bsem = iblks[s]
      pltpu.async_copy(blk_ref(bj), cblk, bsem)

    _for_parity(bj * IBLK, go)

  def wait_blk(bj):
    def go(s):
      cblk, bsem = iblks[s]
      pltpu.make_async_copy(blk_ref(bj), cblk, bsem).wait()

    _for_parity(bj * IBLK, go)

  def fire_gather(ci, b):
    rows, sem = gbufs[b]

    def go(s):
      cblk, _ = iblks[s]
      off = pl.multiple_of(lax.rem(ci, IBLK) * (CHUNK * 2), 8)
      pltpu.async_copy(table_sh.at[cblk.at[pl.ds(off, CHUNK * 2)]], rows, sem)

    _for_parity(ci, go)

  def wait_gather(ci, b):
    rows, sem = gbufs[b]

    def go(s):
      cblk, _ = iblks[s]
      off = pl.multiple_of(lax.rem(ci, IBLK) * (CHUNK * 2), 8)
      pltpu.make_async_copy(table_sh.at[cblk.at[pl.ds(off, CHUNK * 2)]], rows, sem).wait()

    _for_parity(ci, go)

  def compute(ci, b):
    rows, _ = gbufs[b]
    slot = lax.rem(ci, FLUSH)

    def group_body(g, c2):
      eb = g * L
      for j in range(L):
        e = eb + j
        acc = rows[e, pl.ds(0, L)] * rows[CHUNK + e, pl.ds(0, L)]
        for k in range(1, nslice):
          acc = acc + rows[e, pl.ds(k * L, L)] * rows[CHUNK + e, pl.ds(k * L, L)]
        acc_v[pl.ds(j * L, L)] = acc
      lanes = lax.iota(jnp.int32, L) * L
      tot = plsc.load_gather(acc_v, [lanes])
      for i in range(1, L):
        tot = tot + plsc.load_gather(acc_v, [lanes + i])
      out_v[pl.ds(slot * CHUNK + eb, L)] = 1.0 / (1.0 + jnp.exp(-tot))
      return c2

    lax.fori_loop(0, GROUPS, group_body, 0)

  pltpu.sync_copy(blk_ref(0), cblk0)
  fire_blk(1)
  fire_gather(0, 0)

  def outer(io, carry):
    for b in range(2):
      ci = io * 2 + b
      ob = 1 - b

      @pl.when(ci + 1 < n_chunks)
      def _():
        @pl.when(lax.rem(ci + 1, IBLK) == 0)
        def _():
          wait_blk((ci + 1) // IBLK)

        fire_gather(ci + 1, ob)

      wait_gather(ci, b)
      compute(ci, b)

      @pl.when((lax.rem(ci, IBLK) == IBLK - 1) & (ci // IBLK + 2 < n_blocks))
      def _():
        fire_blk(ci // IBLK + 2)

      @pl.when(lax.rem(ci, FLUSH) == FLUSH - 1)
      def _():
        foff = pl.multiple_of(base + (ci - (FLUSH - 1)) * CHUNK, 8)
        pltpu.sync_copy(out_v, out_hbm.at[pl.ds(foff, FLUSH * CHUNK)])

    return carry

  lax.fori_loop(0, n_chunks // 2, outer, 0)


def _link_predict(table, src, tgt):
  E = src.shape[0]
  n_nodes, D = table.shape
  assert E % NW == 0
  per_w = E // NW
  n_chunks = per_w // CHUNK
  assert per_w % CHUNK == 0 and D % L == 0
  assert n_chunks % 2 == 0 and n_chunks % FLUSH == 0 and n_chunks % IBLK == 0
  assert n_nodes % STRIPE == 0 and n_nodes // STRIPE <= NS

  cidx = jnp.stack([src.reshape(-1, CHUNK), tgt.reshape(-1, CHUNK)], axis=1).reshape(-1)

  mesh = plsc.VectorSubcoreMesh(core_axis_name="c", subcore_axis_name="s")
  k = pl.kernel(
      functools.partial(_tec_body, D, per_w, n_nodes),
      out_type=jax.ShapeDtypeStruct((E,), jnp.float32),
      mesh=mesh,
      compiler_params=pltpu.CompilerParams(needs_layout_passes=False),
      scratch_types=[
          pltpu.VMEM_SHARED((n_nodes, D), jnp.float32),
          pltpu.VMEM((IBLK * CHUNK * 2,), jnp.int32),
          pltpu.VMEM((IBLK * CHUNK * 2,), jnp.int32),
          pltpu.VMEM((CHUNK * 2, D), jnp.float32),
          pltpu.VMEM((CHUNK * 2, D), jnp.float32),
          pltpu.VMEM((L * L,), jnp.float32),
          pltpu.VMEM((FLUSH * CHUNK,), jnp.float32),
          pltpu.SemaphoreType.DMA,
          pltpu.SemaphoreType.DMA,
          pltpu.SemaphoreType.DMA,
          pltpu.SemaphoreType.DMA,
      ],
  )
  return k(table, cidx)


def kernel(node_embedding_matrix, pos_edge_index, neg_edge_index, batch_train_x_index):
  src = jnp.concatenate([pos_edge_index[0], neg_edge_index[0]]).astype(jnp.int32)
  tgt = jnp.concatenate([pos_edge_index[1], neg_edge_index[1]]).astype(jnp.int32)
  return _link_predict(node_embedding_matrix, src, tgt)


# speedup vs baseline: 1.1447x; 1.1447x over previous
import functools

import jax
import jax.numpy as jnp
from jax import lax
from jax.experimental import pallas as pl
from jax.experimental.pallas import tpu as pltpu
from jax.experimental.pallas import tpu_sc as plsc

NC = 2
NS = 16
NW = NC * NS
L = 16

CHUNK = 80
GROUPS = CHUNK // L
NBUF = 2
SPLIT = 2
PART = CHUNK // SPLIT


def _tec_body(D, per_w, table_hbm, src_hbm, tgt_hbm, out_hbm,
              sidx_all, tidx_all, srows0, trows0, srows1, trows1,
              acc_v, out_v, sem0, sem1):
  wid = lax.axis_index("s") * NC + lax.axis_index("c")
  n_chunks = per_w // CHUNK
  base = wid * per_w
  nslice = D // L
  bufs = ((srows0, trows0, sem0), (srows1, trows1, sem1))

  pltpu.sync_copy(src_hbm.at[pl.ds(base, per_w)], sidx_all)
  pltpu.sync_copy(tgt_hbm.at[pl.ds(base, per_w)], tidx_all)

  def parts(ci, b):
    srows, trows, sem = bufs[b]
    for h in range(SPLIT):
      off = pl.multiple_of(ci * CHUNK + h * PART, 8)
      dst = pl.ds(h * PART, PART)
      yield table_hbm.at[sidx_all.at[pl.ds(off, PART)]], srows.at[dst], sem
      yield table_hbm.at[tidx_all.at[pl.ds(off, PART)]], trows.at[dst], sem

  def fire(ci, b):
    for s, d, sem in parts(ci, b):
      pltpu.async_copy(s, d, sem)

  def wait(ci, b):
    for s, d, sem in parts(ci, b):
      pltpu.make_async_copy(s, d, sem).wait()

  for b in range(NBUF):
    fire(b, b)

  def compute(ci, srows, trows):
    def group_body(g, c2):
      eb = g * L
      for j in range(L):
        e = eb + j
        acc = srows[e, pl.ds(0, L)] * trows[e, pl.ds(0, L)]
        for k in range(1, nslice):
          acc = acc + srows[e, pl.ds(k * L, L)] * trows[e, pl.ds(k * L, L)]
        acc_v[pl.ds(j * L, L)] = acc
      lanes = lax.iota(jnp.int32, L) * L
      tot = plsc.load_gather(acc_v, [lanes])
      for i in range(1, L):
        tot = tot + plsc.load_gather(acc_v, [lanes + i])
      out_v[pl.ds(ci * CHUNK + eb, L)] = 1.0 / (1.0 + jnp.exp(-tot))
      return c2

    lax.fori_loop(0, GROUPS, group_body, 0)

  def outer(io, carry):
    for b in range(NBUF):
      ci = io * NBUF + b
      srows, trows, _ = bufs[b]
      wait(ci, b)
      compute(ci, srows, trows)

      @pl.when(ci + NBUF < n_chunks)
      def _():
        fire(ci + NBUF, b)

    return carry

  lax.fori_loop(0, n_chunks // NBUF, outer, 0)
  pltpu.sync_copy(out_v, out_hbm.at[pl.ds(wid * per_w, per_w)])


def _link_predict(table, src, tgt):
  E = src.shape[0]
  D = table.shape[1]
  assert E % NW == 0
  per_w = E // NW
  n_chunks = per_w // CHUNK
  assert per_w % CHUNK == 0 and D % L == 0 and n_chunks % NBUF == 0
  assert CHUNK % SPLIT == 0 and PART % 8 == 0

  mesh = plsc.VectorSubcoreMesh(core_axis_name="c", subcore_axis_name="s")
  k = pl.kernel(
      functools.partial(_tec_body, D, per_w),
      out_type=jax.ShapeDtypeStruct((E,), jnp.float32),
      mesh=mesh,
      compiler_params=pltpu.CompilerParams(needs_layout_passes=False),
      scratch_types=[
          pltpu.VMEM((per_w,), jnp.int32),
          pltpu.VMEM((per_w,), jnp.int32),
          pltpu.VMEM((CHUNK, D), jnp.float32),
          pltpu.VMEM((CHUNK, D), jnp.float32),
          pltpu.VMEM((CHUNK, D), jnp.float32),
          pltpu.VMEM((CHUNK, D), jnp.float32),
          pltpu.VMEM((L * L,), jnp.float32),
          pltpu.VMEM((per_w,), jnp.float32),
          pltpu.SemaphoreType.DMA,
          pltpu.SemaphoreType.DMA,
      ],
  )
  return k(table, src, tgt)


def kernel(node_embedding_matrix, pos_edge_index, neg_edge_index, batch_train_x_index):
  src = jnp.concatenate([pos_edge_index[0], neg_edge_index[0]]).astype(jnp.int32)
  tgt = jnp.concatenate([pos_edge_index[1], neg_edge_index[1]]).astype(jnp.int32)
  return _link_predict(node_embedding_matrix, src, tgt)
